# baseline (speedup 1.0000x reference)
import functools
import jax
import jax.numpy as jnp
from jax import lax
from jax.experimental import pallas as pl
from jax.experimental.pallas import tpu as pltpu
from jax.experimental.pallas import tpu_sc as plsc

N_ROWS = 4096
N_COLS = 1000
NW = 32
LPW = N_ROWS // NW
RC = 200
NCH = N_COLS // RC
NV = LPW // 16


def _sc_kernel(p_hbm, t_hbm, out_hbm, pbuf, tbuf, rm, cnt, stage):
    wid = lax.axis_index("s") * 2 + lax.axis_index("c")
    lane0 = wid * LPW

    inf16 = jnp.full((16,), jnp.inf, jnp.float32)
    zero16 = jnp.zeros((16,), jnp.float32)
    one16 = jnp.ones((16,), jnp.float32)
    for v in range(NV):
        rm[pl.ds(v * 16, 16)] = inf16
        cnt[pl.ds(v * 16, 16)] = zero16

    def chunk_min(c, carry):
        pltpu.sync_copy(p_hbm.at[pl.ds(c * RC, RC), pl.ds(lane0, LPW)], pbuf)
        pltpu.sync_copy(t_hbm.at[pl.ds(c * RC, RC), pl.ds(lane0, LPW)], tbuf)

        def row(r, carry2):
            for v in range(NV):
                pv = pbuf[r, pl.ds(v * 16, 16)]
                tv = tbuf[r, pl.ds(v * 16, 16)]
                m = jnp.where(tv > 0.0, pv, inf16)
                rm[pl.ds(v * 16, 16)] = jnp.minimum(rm[pl.ds(v * 16, 16)], m)
            return carry2

        return lax.fori_loop(0, RC, row, carry)

    lax.fori_loop(0, NCH, chunk_min, 0)

    def chunk_cnt(c, carry):
        pltpu.sync_copy(p_hbm.at[pl.ds(c * RC, RC), pl.ds(lane0, LPW)], pbuf)

        def row(r, carry2):
            for v in range(NV):
                pv = pbuf[r, pl.ds(v * 16, 16)]
                ge = pv >= rm[pl.ds(v * 16, 16)]
                cnt[pl.ds(v * 16, 16)] = cnt[pl.ds(v * 16, 16)] + jnp.where(ge, one16, zero16)
            return carry2

        return lax.fori_loop(0, RC, row, carry)

    lax.fori_loop(0, NCH, chunk_cnt, 0)

    total = zero16
    for v in range(NV):
        guarded = jnp.where(rm[pl.ds(v * 16, 16)] < jnp.inf, cnt[pl.ds(v * 16, 16)], zero16)
        total = total + guarded
    stage[...] = total
    pltpu.sync_copy(stage, out_hbm.at[wid])


def kernel(predict_probs, true_labels):
    p = predict_probs.T
    t = true_labels.T
    mesh = plsc.VectorSubcoreMesh(core_axis_name="c", subcore_axis_name="s")
    k = functools.partial(
        pl.kernel,
        mesh=mesh,
        out_type=jax.ShapeDtypeStruct((NW, 16), jnp.float32),
        scratch_types=[
            pltpu.VMEM((RC, LPW), jnp.float32),
            pltpu.VMEM((RC, LPW), jnp.float32),
            pltpu.VMEM((LPW,), jnp.float32),
            pltpu.VMEM((LPW,), jnp.float32),
            pltpu.VMEM((16,), jnp.float32),
        ],
    )(_sc_kernel)
    out = k(p, t)
    return jnp.sum(out) / N_ROWS


# speedup vs baseline: 11.2682x; 11.2682x over previous
import jax
import jax.numpy as jnp
from jax.experimental import pallas as pl
from jax.experimental.pallas import tpu as pltpu

N_ROWS = 4096
N_COLS = 1000
BC = 1024
GRID = N_ROWS // BC


def _cov_kernel(p_ref, t_ref, out_ref):
    p = p_ref[...]
    t = t_ref[...]
    masked = jnp.where(t > 0, p, jnp.inf)
    colmin = jnp.min(masked, axis=0, keepdims=True)
    cov = jnp.sum((p >= colmin).astype(jnp.float32), axis=0)
    cov = jnp.where(jnp.isfinite(colmin[0, :]), cov, 0.0)
    total = jnp.sum(cov)

    i = pl.program_id(0)

    @pl.when(i == 0)
    def _():
        out_ref[0] = 0.0

    out_ref[0] += total

    @pl.when(i == GRID - 1)
    def _():
        out_ref[0] = out_ref[0] * (1.0 / N_ROWS)


def kernel(predict_probs, true_labels):
    p = predict_probs.T
    t = true_labels.T
    out = pl.pallas_call(
        _cov_kernel,
        grid=(GRID,),
        in_specs=[
            pl.BlockSpec((N_COLS, BC), lambda i: (0, i)),
            pl.BlockSpec((N_COLS, BC), lambda i: (0, i)),
        ],
        out_specs=pl.BlockSpec(memory_space=pltpu.SMEM),
        out_shape=jax.ShapeDtypeStruct((1,), jnp.float32),
    )(p, t)
    return out[0]
